# baseline (speedup 1.0000x reference)
import functools

import jax
import jax.numpy as jnp
from jax import lax
from jax.experimental import pallas as pl
from jax.experimental.pallas import tpu as pltpu
from jax.experimental.pallas import tpu_sc as plsc

NC = 2
NS = 16
L = 16
NW = NC * NS


def _embed_body(P, NCHUNK, B, D,
                ids_hbm, wte_hbm, wpe_hbm, out_hbm,
                idx_v, wte_buf, wpe_buf, gsem):
    wid = lax.axis_index("s") * NC + lax.axis_index("c")
    spw = NCHUNK * P
    pos_base = wid * spw

    for b in range(B):
        pltpu.sync_copy(ids_hbm.at[b, wid], idx_v.at[b])

    def chunk(ci, carry):
        pos = pos_base + ci * P
        pltpu.sync_copy(wpe_hbm.at[pl.ds(pos, P)], wpe_buf)
        cps = [
            pltpu.async_copy(wte_hbm.at[idx_v.at[b, ci]], wte_buf.at[b], gsem)
            for b in range(B)
        ]
        for cp in cps:
            cp.wait()
        def grp(g, c):
            col = pl.ds(g * L, L)
            for r in range(P):
                v = wpe_buf[r, col]
                for b in range(B):
                    plsc.addupdate(wte_buf.at[b, r, col], v)
            return c
        lax.fori_loop(0, D // L, grp, 0, unroll=False)
        for b in range(B):
            pltpu.sync_copy(wte_buf.at[b], out_hbm.at[b, pl.ds(pos, P)])
        return carry

    lax.fori_loop(0, NCHUNK, chunk, 0, unroll=False)


def kernel(input_ids, wte, wpe):
    B, S = input_ids.shape
    V, D = wte.shape
    P = 8
    spw = S // NW
    NCHUNK = spw // P

    ids = input_ids.astype(jnp.int32).reshape(B, NW, NCHUNK, P)

    mesh = plsc.VectorSubcoreMesh(
        core_axis_name="c", subcore_axis_name="s",
        num_cores=NC, num_subcores=NS)

    run = pl.kernel(
        functools.partial(_embed_body, P, NCHUNK, B, D),
        out_type=jax.ShapeDtypeStruct((B, S, D), jnp.float32),
        mesh=mesh,
        scratch_types=[
            pltpu.VMEM((B, NCHUNK, P), jnp.int32),
            pltpu.VMEM((B, P, D), jnp.float32),
            pltpu.VMEM((P, D), jnp.float32),
            pltpu.SemaphoreType.DMA,
        ],
    )
    return run(ids, wte, wpe)


# speedup vs baseline: 1.6757x; 1.6757x over previous
import functools

import jax
import jax.numpy as jnp
from jax import lax
from jax.experimental import pallas as pl
from jax.experimental.pallas import tpu as pltpu
from jax.experimental.pallas import tpu_sc as plsc

NC = 2
NS = 16
L = 16
NW = NC * NS
NBUF = 2


def _embed_body(P, NCHUNK, B, D,
                ids_hbm, wte_hbm, wpe_hbm, out_hbm,
                idx_v, wte_buf, wpe_buf, gsem0, gsem1, wsem0, wsem1):
    wid = lax.axis_index("s") * NC + lax.axis_index("c")
    spw = NCHUNK * P
    pos_base = wid * spw
    gsems = [gsem0, gsem1]
    wsems = [wsem0, wsem1]

    for b in range(B):
        pltpu.sync_copy(ids_hbm.at[b, wid], idx_v.at[b])

    def in_copies(ci, j):
        pos = pos_base + ci * P
        cps = [pltpu.make_async_copy(
            wpe_hbm.at[pl.ds(pos, P)], wpe_buf.at[j], gsems[j])]
        for b in range(B):
            cps.append(pltpu.make_async_copy(
                wte_hbm.at[idx_v.at[b, ci]], wte_buf.at[j, b], gsems[j]))
        return cps

    def out_copies(ci, j):
        pos = pos_base + ci * P
        return [pltpu.make_async_copy(
            wte_buf.at[j, b], out_hbm.at[b, pl.ds(pos, P)], wsems[j])
            for b in range(B)]

    def compute(j):
        def grp(g, c):
            col = pl.ds(g * L, L)
            for r in range(P):
                v = wpe_buf[j, r, col]
                for b in range(B):
                    plsc.addupdate(wte_buf.at[j, b, r, col], v)
            return c
        lax.fori_loop(0, D // L, grp, 0, unroll=False)

    for cp in in_copies(0, 0):
        cp.start()

    def outer(o, carry):
        for j in range(NBUF):
            ci = o * NBUF + j
            nj = (j + 1) % NBUF

            @pl.when(ci + 1 < NCHUNK)
            def _prefetch():
                @pl.when(ci >= 1)
                def _drain():
                    for cp in out_copies(ci - 1, nj):
                        cp.wait()
                for cp in in_copies(ci + 1, nj):
                    cp.start()

            for cp in in_copies(ci, j):
                cp.wait()
            compute(j)
            for cp in out_copies(ci, j):
                cp.start()
        return carry

    lax.fori_loop(0, NCHUNK // NBUF, outer, 0, unroll=False)

    for k in range(NBUF):
        ci = NCHUNK - NBUF + k
        for cp in out_copies(ci, ci % NBUF):
            cp.wait()


def kernel(input_ids, wte, wpe):
    B, S = input_ids.shape
    V, D = wte.shape
    P = 8
    spw = S // NW
    NCHUNK = spw // P

    ids = input_ids.astype(jnp.int32).reshape(B, NW, NCHUNK, P)

    mesh = plsc.VectorSubcoreMesh(
        core_axis_name="c", subcore_axis_name="s",
        num_cores=NC, num_subcores=NS)

    run = pl.kernel(
        functools.partial(_embed_body, P, NCHUNK, B, D),
        out_type=jax.ShapeDtypeStruct((B, S, D), jnp.float32),
        mesh=mesh,
        scratch_types=[
            pltpu.VMEM((B, NCHUNK, P), jnp.int32),
            pltpu.VMEM((NBUF, B, P, D), jnp.float32),
            pltpu.VMEM((NBUF, P, D), jnp.float32),
            pltpu.SemaphoreType.DMA,
            pltpu.SemaphoreType.DMA,
            pltpu.SemaphoreType.DMA,
            pltpu.SemaphoreType.DMA,
        ],
    )
    return run(ids, wte, wpe)


# speedup vs baseline: 1.8042x; 1.0766x over previous
import functools

import jax
import jax.numpy as jnp
from jax import lax
from jax.experimental import pallas as pl
from jax.experimental.pallas import tpu as pltpu
from jax.experimental.pallas import tpu_sc as plsc

NC = 2
NS = 16
L = 16
NW = NC * NS
NBUF = 4


def _embed_body(P, NCHUNK, B, D,
                ids_hbm, wte_hbm, wpe_hbm, out_hbm,
                idx_v, wte_buf, wpe_buf, *sems):
    wid = lax.axis_index("s") * NC + lax.axis_index("c")
    spw = NCHUNK * P
    pos_base = wid * spw
    gsems = sems[:NBUF]
    wsems = sems[NBUF:]

    for b in range(B):
        pltpu.sync_copy(ids_hbm.at[b, wid], idx_v.at[b])

    def in_copies(ci, j):
        pos = pos_base + ci * P
        cps = [pltpu.make_async_copy(
            wpe_hbm.at[pl.ds(pos, P)], wpe_buf.at[j], gsems[j])]
        for b in range(B):
            cps.append(pltpu.make_async_copy(
                wte_hbm.at[idx_v.at[b, ci]], wte_buf.at[j, b], gsems[j]))
        return cps

    def out_copies(ci, j):
        pos = pos_base + ci * P
        return [pltpu.make_async_copy(
            wte_buf.at[j, b], out_hbm.at[b, pl.ds(pos, P)], wsems[j])
            for b in range(B)]

    def compute(j):
        def grp(g, c):
            col = pl.ds(g * L, L)
            for r in range(P):
                v = wpe_buf[j, r, col]
                for b in range(B):
                    plsc.addupdate(wte_buf.at[j, b, r, col], v)
            return c
        lax.fori_loop(0, D // L, grp, 0, unroll=False)

    for cp in in_copies(0, 0):
        cp.start()

    def outer(o, carry):
        for j in range(NBUF):
            ci = o * NBUF + j
            nj = (j + 1) % NBUF

            @pl.when(ci + 1 < NCHUNK)
            def _prefetch():
                @pl.when(ci >= NBUF - 1)
                def _drain():
                    for cp in out_copies(ci - (NBUF - 1), nj):
                        cp.wait()
                for cp in in_copies(ci + 1, nj):
                    cp.start()

            for cp in in_copies(ci, j):
                cp.wait()
            compute(j)
            for cp in out_copies(ci, j):
                cp.start()
        return carry

    lax.fori_loop(0, NCHUNK // NBUF, outer, 0, unroll=False)

    for k in range(NBUF):
        ci = NCHUNK - NBUF + k
        for cp in out_copies(ci, ci % NBUF):
            cp.wait()


def kernel(input_ids, wte, wpe):
    B, S = input_ids.shape
    V, D = wte.shape
    P = 4
    spw = S // NW
    NCHUNK = spw // P

    ids = input_ids.astype(jnp.int32).reshape(B, NW, NCHUNK, P)

    mesh = plsc.VectorSubcoreMesh(
        core_axis_name="c", subcore_axis_name="s",
        num_cores=NC, num_subcores=NS)

    run = pl.kernel(
        functools.partial(_embed_body, P, NCHUNK, B, D),
        out_type=jax.ShapeDtypeStruct((B, S, D), jnp.float32),
        mesh=mesh,
        scratch_types=[
            pltpu.VMEM((B, NCHUNK, P), jnp.int32),
            pltpu.VMEM((NBUF, B, P, D), jnp.float32),
            pltpu.VMEM((NBUF, P, D), jnp.float32),
        ] + [pltpu.SemaphoreType.DMA] * (2 * NBUF),
    )
    return run(ids, wte, wpe)
